# baseline (speedup 1.0000x reference)
import functools

import jax
import jax.numpy as jnp
from jax import lax
from jax.experimental import pallas as pl
from jax.experimental.pallas import tpu as pltpu
from jax.experimental.pallas import tpu_sc as plsc

N = 10000
E = 160000
NACC = 10112
F32 = jnp.float32


def _seg_sum_make(Cw, NCH):
    assert NCH % 14 == 0
    nbody = NCH // 14
    mesh = plsc.VectorSubcoreMesh(
        core_axis_name="c", subcore_axis_name="s", num_cores=2, num_subcores=16
    )

    @functools.partial(
        pl.kernel,
        out_type=jax.ShapeDtypeStruct((2 * NACC, Cw), F32),
        mesh=mesh,
        scratch_types=[
            pltpu.VMEM((7, 128), jnp.int32),
            pltpu.VMEM((7, 128), jnp.int32),
            pltpu.VMEM((7, 128), jnp.int32),
            pltpu.VMEM((7, 128), jnp.int32),
            pltpu.VMEM((128, Cw), F32),
            pltpu.VMEM((128, Cw), F32),
            pltpu.VMEM_SHARED((NACC, Cw), F32),
            pltpu.SemaphoreType.DMA,
            pltpu.SemaphoreType.DMA,
            pltpu.SemaphoreType.DMA,
            pltpu.SemaphoreType.DMA,
            pltpu.SemaphoreType.DMA,
            pltpu.SemaphoreType.DMA,
        ],
    )
    def k(y_hbm, srcp_hbm, dstp_hbm, zeros_hbm, out_hbm,
          sblk0, sblk1, dblk0, dblk1, buf_a, buf_b, acc,
          sem_g, sem_sc, sem_s0, sem_s1, sem_d0, sem_d1):
        c = lax.axis_index("c")
        s = lax.axis_index("s")
        w = c * 16 + s
        bufs = (buf_a, buf_b)
        sblks = (sblk0, sblk1)
        dblks = (dblk0, dblk1)
        ssems = (sem_s0, sem_s1)
        dsems = (sem_d0, sem_d1)
        pltpu.sync_copy(zeros_hbm.at[pl.ds(s * 632, 632)], acc.at[pl.ds(s * 632, 632)])
        plsc.subcore_barrier()

        def gather(idx_ref, buf):
            pltpu.async_copy(y_hbm.at[idx_ref], buf, sem_g)

        def wait_g(buf):
            pltpu.make_async_copy(y_hbm.at[sblk0.at[0]], buf, sem_g).wait()

        def sload(blk, slot):
            pltpu.async_copy(srcp_hbm.at[w, blk], sblks[slot], ssems[slot])

        def wait_si(slot):
            pltpu.make_async_copy(srcp_hbm.at[w, 0], sblks[slot], ssems[slot]).wait()

        def dload(blk, slot):
            pltpu.async_copy(dstp_hbm.at[w, blk], dblks[slot], dsems[slot])

        def wait_d(slot):
            pltpu.make_async_copy(dstp_hbm.at[w, 0], dblks[slot], dsems[slot]).wait()

        def half(slot):
            wait_si(slot)
            for r in (0, 2, 4):
                cpa = pltpu.async_copy(y_hbm.at[sblks[slot].at[r]], buf_a, sem_g)
                cpb = pltpu.async_copy(y_hbm.at[sblks[slot].at[r + 1]], buf_b, sem_g)
                cpa.wait()
                cpb.wait()
                if r == 0:
                    wait_d(slot)
                sca = pltpu.async_copy(buf_a, acc.at[dblks[slot].at[r]], sem_sc, add=True)
                scb = pltpu.async_copy(buf_b, acc.at[dblks[slot].at[r + 1]], sem_sc, add=True)
                sca.wait()
                scb.wait()
            cpa = pltpu.async_copy(y_hbm.at[sblks[slot].at[6]], buf_a, sem_g)
            cpa.wait()
            pltpu.sync_copy(buf_a, acc.at[dblks[slot].at[6]], add=True)

        sload(0, 0)
        dload(0, 0)

        def body(i2, carry):
            sload(2 * i2 + 1, 1)
            dload(2 * i2 + 1, 1)
            half(0)

            @pl.when(i2 < nbody - 1)
            def _():
                sload(2 * i2 + 2, 0)
                dload(2 * i2 + 2, 0)

            half(1)
            return carry

        lax.fori_loop(0, nbody, body, 0)
        plsc.subcore_barrier()
        pltpu.sync_copy(
            acc.at[pl.ds(s * 632, 632)], out_hbm.at[pl.ds(c * NACC + s * 632, 632)]
        )

    return k


_RB = 1000


def _matmul_split(x, W):
    K = x.shape[1]

    def body(x_ref, w_ref, o_ref):
        o_ref[...] = jnp.dot(x_ref[...], w_ref[...], preferred_element_type=F32)

    return pl.pallas_call(
        body,
        grid=(2, N // _RB),
        in_specs=[
            pl.BlockSpec((_RB, K), lambda j, i: (i, 0)),
            pl.BlockSpec((K, 128), lambda j, i: (0, j)),
        ],
        out_specs=pl.BlockSpec((_RB, 128), lambda j, i: (j * (N // _RB) + i, 0)),
        out_shape=jax.ShapeDtypeStruct((2 * N, 128), F32),
    )(x, W)


def _matmul_plain(x, W):
    K = x.shape[1]
    Co = W.shape[1]

    def body(x_ref, w_ref, o_ref):
        o_ref[...] = jnp.dot(x_ref[...], w_ref[...], preferred_element_type=F32)

    return pl.pallas_call(
        body,
        grid=(N // _RB,),
        in_specs=[
            pl.BlockSpec((_RB, K), lambda i: (i, 0)),
            pl.BlockSpec((K, Co), lambda i: (0, 0)),
        ],
        out_specs=pl.BlockSpec((_RB, Co), lambda i: (i, 0)),
        out_shape=jax.ShapeDtypeStruct((N, Co), F32),
    )(x, W)


def _mlp_wide(aggflat, b, M1, mb1, M2, mb2, final_softplus):
    agg3 = aggflat.reshape(2, NACC, 128)
    b2 = b.reshape(2, 128)
    M13 = M1.reshape(2, 128, 512)
    mb1r = mb1.reshape(1, 512)
    mb2r = mb2.reshape(1, 256)

    def body(a0_ref, a1_ref, b_ref, m1_ref, mb1_ref, m2_ref, mb2_ref, o_ref):
        h0 = jnp.maximum(a0_ref[0] + b_ref[0], 0.0)
        h1 = jnp.maximum(a1_ref[0] + b_ref[1], 0.0)
        t = jnp.dot(h0, m1_ref[0], preferred_element_type=F32)
        t += jnp.dot(h1, m1_ref[1], preferred_element_type=F32)
        t = jnp.maximum(t + mb1_ref[0], 0.0)
        o = jnp.dot(t, m2_ref[...], preferred_element_type=F32) + mb2_ref[0]
        if final_softplus:
            o = 10.0 * o
            o = (jnp.maximum(o, 0.0) + jnp.log1p(jnp.exp(-jnp.abs(o)))) / 10.0
        o_ref[...] = o

    return pl.pallas_call(
        body,
        grid=(N // _RB,),
        in_specs=[
            pl.BlockSpec((1, _RB, 128), lambda i: (0, i, 0)),
            pl.BlockSpec((1, _RB, 128), lambda i: (1, i, 0)),
            pl.BlockSpec((2, 128), lambda i: (0, 0)),
            pl.BlockSpec((2, 128, 512), lambda i: (0, 0, 0)),
            pl.BlockSpec((1, 512), lambda i: (0, 0)),
            pl.BlockSpec((512, 256), lambda i: (0, 0)),
            pl.BlockSpec((1, 256), lambda i: (0, 0)),
        ],
        out_specs=pl.BlockSpec((_RB, 256), lambda i: (i, 0)),
        out_shape=jax.ShapeDtypeStruct((N, 256), F32),
    )(agg3, agg3, b2, M13, mb1r, M2, mb2r)


def _mlp_narrow(partflat, b, M1, mb1, M2, mb2):
    p3 = partflat.reshape(2, NACC, 128)
    br = b.reshape(1, 32)
    mb1r = mb1.reshape(1, 64)
    mb2r = mb2.reshape(1, 32)

    def body(p0_ref, p1_ref, b_ref, m1_ref, mb1_ref, m2_ref, mb2_ref, o_ref):
        a = (p0_ref[0] + p1_ref[0])[:, :32]
        h = jnp.maximum(a + b_ref[0], 0.0)
        t = jnp.maximum(jnp.dot(h, m1_ref[...], preferred_element_type=F32) + mb1_ref[0], 0.0)
        o_ref[...] = jnp.dot(t, m2_ref[...], preferred_element_type=F32) + mb2_ref[0]

    return pl.pallas_call(
        body,
        grid=(N // _RB,),
        in_specs=[
            pl.BlockSpec((1, _RB, 128), lambda i: (0, i, 0)),
            pl.BlockSpec((1, _RB, 128), lambda i: (1, i, 0)),
            pl.BlockSpec((1, 32), lambda i: (0, 0)),
            pl.BlockSpec((32, 64), lambda i: (0, 0)),
            pl.BlockSpec((1, 64), lambda i: (0, 0)),
            pl.BlockSpec((64, 32), lambda i: (0, 0)),
            pl.BlockSpec((1, 32), lambda i: (0, 0)),
        ],
        out_specs=pl.BlockSpec((_RB, 32), lambda i: (i, 0)),
        out_shape=jax.ShapeDtypeStruct((N, 32), F32),
    )(p3, p3, br, M1, mb1r, M2, mb2r)


def _fc_enc(hflat, W, b):
    KB = 32000

    def body(x_ref, w_ref, b_ref, o_ref):
        @pl.when(pl.program_id(0) == 0)
        def _():
            o_ref[...] = b_ref[...]

        o_ref[...] += jnp.dot(x_ref[...], w_ref[...], preferred_element_type=F32)

    return pl.pallas_call(
        body,
        grid=(10,),
        in_specs=[
            pl.BlockSpec((1, KB), lambda i: (0, i)),
            pl.BlockSpec((KB, 64), lambda i: (i, 0)),
            pl.BlockSpec((1, 64), lambda i: (0, 0)),
        ],
        out_specs=pl.BlockSpec((1, 64), lambda i: (0, 0)),
        out_shape=jax.ShapeDtypeStruct((1, 64), F32),
    )(hflat, W, b)


def _fc_dec(z, W, b):
    NB = 32000

    def body(z_ref, w_ref, b_ref, o_ref):
        o_ref[...] = jnp.dot(z_ref[...], w_ref[...], preferred_element_type=F32) + b_ref[...]

    return pl.pallas_call(
        body,
        grid=(10,),
        in_specs=[
            pl.BlockSpec((1, 64), lambda i: (0, 0)),
            pl.BlockSpec((64, NB), lambda i: (0, i)),
            pl.BlockSpec((1, NB), lambda i: (0, i)),
        ],
        out_specs=pl.BlockSpec((1, NB), lambda i: (0, i)),
        out_shape=jax.ShapeDtypeStruct((1, 320000), F32),
    )(z, W, b)


def kernel(x, pos, enc_params, dec_params, fc_e_W, fc_e_b, fc_d_W, fc_d_b, edge_index):
    del pos
    src = edge_index[0]
    dst = edge_index[1]
    i32 = jnp.int32

    srcw = jnp.concatenate(
        [src.reshape(16, 10000), jnp.zeros((16, 84 * 128 - 10000), i32)], axis=1
    ).reshape(16, 84, 128)
    srcw2 = jnp.concatenate([srcw, srcw + N], axis=0).reshape(32, 12, 7, 128)
    dstw = jnp.concatenate(
        [dst.reshape(16, 10000), jnp.full((16, 84 * 128 - 10000), N, i32)], axis=1
    ).reshape(16, 84, 128)
    dstw2 = jnp.concatenate([dstw, dstw], axis=0).reshape(32, 12, 7, 128)
    srcn = jnp.concatenate(
        [src.reshape(32, 5000), jnp.zeros((32, 42 * 128 - 5000), i32)], axis=1
    ).reshape(32, 42, 128).reshape(32, 6, 7, 128)
    dstn = jnp.concatenate(
        [dst.reshape(32, 5000), jnp.full((32, 42 * 128 - 5000), N, i32)], axis=1
    ).reshape(32, 42, 128).reshape(32, 6, 7, 128)

    zeros_w = jnp.zeros((NACC, 128), F32)
    seg_wide = _seg_sum_make(128, 84)
    seg_narrow = _seg_sum_make(128, 42)

    def layer(h, params, final_softplus=False):
        W, b, M1, mb1, M2, mb2 = params
        if W.shape[1] == 256:
            yflat = _matmul_split(h, W)
            aggflat = seg_wide(yflat, srcw2, dstw2, zeros_w)
            return _mlp_wide(aggflat, b, M1, mb1, M2, mb2, final_softplus)
        else:
            Wp = jnp.concatenate([W, jnp.zeros((W.shape[0], 96), F32)], axis=1)
            y = _matmul_plain(h, Wp)
            partflat = seg_narrow(y, srcn, dstn, zeros_w)
            return _mlp_narrow(partflat, b, M1, mb1, M2, mb2)

    h = x
    for p in enc_params:
        h = layer(h, p)
    z = _fc_enc(h.reshape(1, 32 * N), fc_e_W, fc_e_b.reshape(1, 64))
    d = _fc_dec(z, fc_d_W, fc_d_b.reshape(1, 320000)).reshape(N, 32)
    for li, p in enumerate(dec_params):
        d = layer(d, p, final_softplus=(li == 2))
    return d


# speedup vs baseline: 2.2008x; 2.2008x over previous
import functools

import jax
import jax.numpy as jnp
from jax import lax
from jax.experimental import pallas as pl
from jax.experimental.pallas import tpu as pltpu
from jax.experimental.pallas import tpu_sc as plsc

N = 10000
E = 160000
NACC = 10240
F32 = jnp.float32


def _seg_sum_make(Cw, NCH):
    mesh = plsc.VectorSubcoreMesh(
        core_axis_name="c", subcore_axis_name="s", num_cores=2, num_subcores=16
    )

    @functools.partial(
        pl.kernel,
        out_type=jax.ShapeDtypeStruct((2 * NACC, Cw), F32),
        mesh=mesh,
        scratch_types=[
            pltpu.VMEM((NCH, 128), jnp.int32),
            pltpu.VMEM((NCH, 128), jnp.int32),
            pltpu.VMEM((128, Cw), F32),
            pltpu.VMEM_SHARED((NACC, Cw), F32),
            pltpu.SemaphoreType.DMA,
        ],
    )
    def k(y_hbm, srcp_hbm, dstp_hbm, zeros_hbm, out_hbm, src_v, dst_v, rows_v, acc, sem):
        c = lax.axis_index("c")
        s = lax.axis_index("s")
        w = c * 16 + s
        pltpu.sync_copy(srcp_hbm.at[w], src_v)
        pltpu.sync_copy(dstp_hbm.at[w], dst_v)
        pltpu.sync_copy(zeros_hbm.at[pl.ds(s * 640, 640)], acc.at[pl.ds(s * 640, 640)])
        plsc.subcore_barrier()

        def chunk(j, carry):
            pltpu.async_copy(y_hbm.at[src_v.at[j]], rows_v, sem).wait()
            pltpu.sync_copy(rows_v, acc.at[dst_v.at[j]], add=True)
            return carry

        lax.fori_loop(0, NCH, chunk, 0)
        plsc.subcore_barrier()
        pltpu.sync_copy(
            acc.at[pl.ds(s * 640, 640)], out_hbm.at[pl.ds(c * NACC + s * 640, 640)]
        )

    return k


_RB = 1000


def _matmul_split(x, W):
    K = x.shape[1]

    def body(x_ref, w_ref, o_ref):
        o_ref[...] = jnp.dot(x_ref[...], w_ref[...], preferred_element_type=F32)

    return pl.pallas_call(
        body,
        grid=(2, N // _RB),
        in_specs=[
            pl.BlockSpec((_RB, K), lambda j, i: (i, 0)),
            pl.BlockSpec((K, 128), lambda j, i: (0, j)),
        ],
        out_specs=pl.BlockSpec((_RB, 128), lambda j, i: (j * (N // _RB) + i, 0)),
        out_shape=jax.ShapeDtypeStruct((2 * N, 128), F32),
    )(x, W)


def _matmul_plain(x, W):
    K = x.shape[1]
    Co = W.shape[1]

    def body(x_ref, w_ref, o_ref):
        o_ref[...] = jnp.dot(x_ref[...], w_ref[...], preferred_element_type=F32)

    return pl.pallas_call(
        body,
        grid=(N // _RB,),
        in_specs=[
            pl.BlockSpec((_RB, K), lambda i: (i, 0)),
            pl.BlockSpec((K, Co), lambda i: (0, 0)),
        ],
        out_specs=pl.BlockSpec((_RB, Co), lambda i: (i, 0)),
        out_shape=jax.ShapeDtypeStruct((N, Co), F32),
    )(x, W)


def _mlp_wide(aggflat, b, M1, mb1, M2, mb2, final_softplus):
    agg3 = aggflat.reshape(2, NACC, 128)
    b2 = b.reshape(2, 128)
    M13 = M1.reshape(2, 128, 512)
    mb1r = mb1.reshape(1, 512)
    mb2r = mb2.reshape(1, 256)

    def body(a0_ref, a1_ref, b_ref, m1_ref, mb1_ref, m2_ref, mb2_ref, o_ref):
        h0 = jnp.maximum(a0_ref[0] + b_ref[0], 0.0)
        h1 = jnp.maximum(a1_ref[0] + b_ref[1], 0.0)
        t = jnp.dot(h0, m1_ref[0], preferred_element_type=F32)
        t += jnp.dot(h1, m1_ref[1], preferred_element_type=F32)
        t = jnp.maximum(t + mb1_ref[0], 0.0)
        o = jnp.dot(t, m2_ref[...], preferred_element_type=F32) + mb2_ref[0]
        if final_softplus:
            o = 10.0 * o
            o = (jnp.maximum(o, 0.0) + jnp.log1p(jnp.exp(-jnp.abs(o)))) / 10.0
        o_ref[...] = o

    return pl.pallas_call(
        body,
        grid=(N // _RB,),
        in_specs=[
            pl.BlockSpec((1, _RB, 128), lambda i: (0, i, 0)),
            pl.BlockSpec((1, _RB, 128), lambda i: (1, i, 0)),
            pl.BlockSpec((2, 128), lambda i: (0, 0)),
            pl.BlockSpec((2, 128, 512), lambda i: (0, 0, 0)),
            pl.BlockSpec((1, 512), lambda i: (0, 0)),
            pl.BlockSpec((512, 256), lambda i: (0, 0)),
            pl.BlockSpec((1, 256), lambda i: (0, 0)),
        ],
        out_specs=pl.BlockSpec((_RB, 256), lambda i: (i, 0)),
        out_shape=jax.ShapeDtypeStruct((N, 256), F32),
    )(agg3, agg3, b2, M13, mb1r, M2, mb2r)


def _mlp_narrow(partflat, b, M1, mb1, M2, mb2):
    p3 = partflat.reshape(2, NACC, 128)
    br = b.reshape(1, 32)
    mb1r = mb1.reshape(1, 64)
    mb2r = mb2.reshape(1, 32)

    def body(p0_ref, p1_ref, b_ref, m1_ref, mb1_ref, m2_ref, mb2_ref, o_ref):
        a = (p0_ref[0] + p1_ref[0])[:, :32]
        h = jnp.maximum(a + b_ref[0], 0.0)
        t = jnp.maximum(jnp.dot(h, m1_ref[...], preferred_element_type=F32) + mb1_ref[0], 0.0)
        o_ref[...] = jnp.dot(t, m2_ref[...], preferred_element_type=F32) + mb2_ref[0]

    return pl.pallas_call(
        body,
        grid=(N // _RB,),
        in_specs=[
            pl.BlockSpec((1, _RB, 128), lambda i: (0, i, 0)),
            pl.BlockSpec((1, _RB, 128), lambda i: (1, i, 0)),
            pl.BlockSpec((1, 32), lambda i: (0, 0)),
            pl.BlockSpec((32, 64), lambda i: (0, 0)),
            pl.BlockSpec((1, 64), lambda i: (0, 0)),
            pl.BlockSpec((64, 32), lambda i: (0, 0)),
            pl.BlockSpec((1, 32), lambda i: (0, 0)),
        ],
        out_specs=pl.BlockSpec((_RB, 32), lambda i: (i, 0)),
        out_shape=jax.ShapeDtypeStruct((N, 32), F32),
    )(p3, p3, br, M1, mb1r, M2, mb2r)


def _fc_enc(hflat, W, b):
    KB = 32000

    def body(x_ref, w_ref, b_ref, o_ref):
        @pl.when(pl.program_id(0) == 0)
        def _():
            o_ref[...] = b_ref[...]

        o_ref[...] += jnp.dot(x_ref[...], w_ref[...], preferred_element_type=F32)

    return pl.pallas_call(
        body,
        grid=(10,),
        in_specs=[
            pl.BlockSpec((1, KB), lambda i: (0, i)),
            pl.BlockSpec((KB, 64), lambda i: (i, 0)),
            pl.BlockSpec((1, 64), lambda i: (0, 0)),
        ],
        out_specs=pl.BlockSpec((1, 64), lambda i: (0, 0)),
        out_shape=jax.ShapeDtypeStruct((1, 64), F32),
    )(hflat, W, b)


def _fc_dec(z, W, b):
    NB = 32000

    def body(z_ref, w_ref, b_ref, o_ref):
        o_ref[...] = jnp.dot(z_ref[...], w_ref[...], preferred_element_type=F32) + b_ref[...]

    return pl.pallas_call(
        body,
        grid=(10,),
        in_specs=[
            pl.BlockSpec((1, 64), lambda i: (0, 0)),
            pl.BlockSpec((64, NB), lambda i: (0, i)),
            pl.BlockSpec((1, NB), lambda i: (0, i)),
        ],
        out_specs=pl.BlockSpec((1, NB), lambda i: (0, i)),
        out_shape=jax.ShapeDtypeStruct((1, 320000), F32),
    )(z, W, b)


def kernel(x, pos, enc_params, dec_params, fc_e_W, fc_e_b, fc_d_W, fc_d_b, edge_index):
    del pos
    src = edge_index[0]
    dst = edge_index[1]
    i32 = jnp.int32

    srcw = jnp.concatenate(
        [src.reshape(16, 10000), jnp.zeros((16, 112), i32)], axis=1
    ).reshape(16, 79, 128)
    srcw2 = jnp.concatenate([srcw, srcw + N], axis=0)
    dstw = jnp.concatenate(
        [dst.reshape(16, 10000), jnp.full((16, 112), N, i32)], axis=1
    ).reshape(16, 79, 128)
    dstw2 = jnp.concatenate([dstw, dstw], axis=0)
    srcn = jnp.concatenate(
        [src.reshape(32, 5000), jnp.zeros((32, 120), i32)], axis=1
    ).reshape(32, 40, 128)
    dstn = jnp.concatenate(
        [dst.reshape(32, 5000), jnp.full((32, 120), N, i32)], axis=1
    ).reshape(32, 40, 128)

    zeros_w = jnp.zeros((NACC, 128), F32)
    seg_wide = _seg_sum_make(128, 79)
    seg_narrow = _seg_sum_make(128, 40)

    def layer(h, params, final_softplus=False):
        W, b, M1, mb1, M2, mb2 = params
        if W.shape[1] == 256:
            yflat = _matmul_split(h, W)
            aggflat = seg_wide(yflat, srcw2, dstw2, zeros_w)
            return _mlp_wide(aggflat, b, M1, mb1, M2, mb2, final_softplus)
        else:
            Wp = jnp.concatenate([W, jnp.zeros((W.shape[0], 96), F32)], axis=1)
            y = _matmul_plain(h, Wp)
            partflat = seg_narrow(y, srcn, dstn, zeros_w)
            return _mlp_narrow(partflat, b, M1, mb1, M2, mb2)

    h = x
    for p in enc_params:
        h = layer(h, p)
    z = _fc_enc(h.reshape(1, 32 * N), fc_e_W, fc_e_b.reshape(1, 64))
    d = _fc_dec(z, fc_d_W, fc_d_b.reshape(1, 320000)).reshape(N, 32)
    for li, p in enumerate(dec_params):
        d = layer(d, p, final_softplus=(li == 2))
    return d


# speedup vs baseline: 2.2902x; 1.0406x over previous
import functools

import jax
import jax.numpy as jnp
from jax import lax
from jax.experimental import pallas as pl
from jax.experimental.pallas import tpu as pltpu
from jax.experimental.pallas import tpu_sc as plsc

N = 10000
E = 160000
NACC = 10240
F32 = jnp.float32


def _seg_sum_make(Cw, NCH):
    mesh = plsc.VectorSubcoreMesh(
        core_axis_name="c", subcore_axis_name="s", num_cores=2, num_subcores=16
    )

    @functools.partial(
        pl.kernel,
        out_type=jax.ShapeDtypeStruct((2 * NACC, Cw), F32),
        mesh=mesh,
        scratch_types=[
            pltpu.VMEM((NCH, 128), jnp.int32),
            pltpu.VMEM((NCH, 128), jnp.int32),
            pltpu.VMEM((128, Cw), F32),
            pltpu.VMEM_SHARED((NACC, Cw), F32),
            pltpu.SemaphoreType.DMA,
        ],
    )
    def k(y_hbm, srcp_hbm, dstp_hbm, zeros_hbm, out_hbm, src_v, dst_v, rows_v, acc, sem):
        c = lax.axis_index("c")
        s = lax.axis_index("s")
        w = c * 16 + s
        pltpu.sync_copy(srcp_hbm.at[w], src_v)
        pltpu.sync_copy(dstp_hbm.at[w], dst_v)
        pltpu.sync_copy(zeros_hbm.at[pl.ds(s * 640, 640)], acc.at[pl.ds(s * 640, 640)])
        plsc.subcore_barrier()

        def chunk(j, carry):
            pltpu.async_copy(y_hbm.at[src_v.at[j]], rows_v, sem).wait()
            pltpu.sync_copy(rows_v, acc.at[dst_v.at[j]], add=True)
            return carry

        lax.fori_loop(0, NCH, chunk, 0)
        plsc.subcore_barrier()
        pltpu.sync_copy(
            acc.at[pl.ds(s * 640, 640)], out_hbm.at[pl.ds(c * NACC + s * 640, 640)]
        )

    return k


_RB = 1000


def _matmul_split(x, W):
    K = x.shape[1]

    def body(x_ref, w_ref, o_ref):
        o_ref[...] = jnp.dot(x_ref[...], w_ref[...], preferred_element_type=F32)

    return pl.pallas_call(
        body,
        grid=(2, N // _RB),
        in_specs=[
            pl.BlockSpec((_RB, K), lambda j, i: (i, 0)),
            pl.BlockSpec((K, 128), lambda j, i: (0, j)),
        ],
        out_specs=pl.BlockSpec((_RB, 128), lambda j, i: (j * (N // _RB) + i, 0)),
        out_shape=jax.ShapeDtypeStruct((2 * N, 128), F32),
    )(x, W)


def _matmul_plain(x, W):
    K = x.shape[1]
    Co = W.shape[1]

    def body(x_ref, w_ref, o_ref):
        o_ref[...] = jnp.dot(x_ref[...], w_ref[...], preferred_element_type=F32)

    return pl.pallas_call(
        body,
        grid=(N // _RB,),
        in_specs=[
            pl.BlockSpec((_RB, K), lambda i: (i, 0)),
            pl.BlockSpec((K, Co), lambda i: (0, 0)),
        ],
        out_specs=pl.BlockSpec((_RB, Co), lambda i: (i, 0)),
        out_shape=jax.ShapeDtypeStruct((N, Co), F32),
    )(x, W)


def _mlp_wide(aggflat, b, M1, mb1, M2, mb2, final_softplus, W_next=None):
    agg3 = aggflat.reshape(2, NACC, 128)
    b2 = b.reshape(2, 128)
    M13 = M1.reshape(2, 128, 512)
    mb1r = mb1.reshape(1, 512)
    mb2r = mb2.reshape(1, 256)
    fuse_split = W_next is not None and W_next.shape[1] == 256
    if fuse_split:
        wn = W_next.reshape(1, 256, 256)
        out_spec = pl.BlockSpec((2, _RB, 128), lambda i: (0, i, 0))
        out_shape = jax.ShapeDtypeStruct((2, N, 128), F32)
    elif W_next is not None:
        wn = W_next.reshape(1, 256, W_next.shape[1])
        out_spec = pl.BlockSpec((_RB, W_next.shape[1]), lambda i: (i, 0))
        out_shape = jax.ShapeDtypeStruct((N, W_next.shape[1]), F32)
    else:
        wn = jnp.zeros((1, 1, 1), F32)
        out_spec = pl.BlockSpec((_RB, 256), lambda i: (i, 0))
        out_shape = jax.ShapeDtypeStruct((N, 256), F32)

    def body(a0_ref, a1_ref, b_ref, m1_ref, mb1_ref, m2_ref, mb2_ref, wn_ref, o_ref):
        h0 = jnp.maximum(a0_ref[0] + b_ref[0], 0.0)
        h1 = jnp.maximum(a1_ref[0] + b_ref[1], 0.0)
        t = jnp.dot(h0, m1_ref[0], preferred_element_type=F32)
        t += jnp.dot(h1, m1_ref[1], preferred_element_type=F32)
        t = jnp.maximum(t + mb1_ref[0], 0.0)
        o = jnp.dot(t, m2_ref[...], preferred_element_type=F32) + mb2_ref[0]
        if final_softplus:
            o = 10.0 * o
            o = (jnp.maximum(o, 0.0) + jnp.log1p(jnp.exp(-jnp.abs(o)))) / 10.0
        if fuse_split:
            y = jnp.dot(o, wn_ref[0], preferred_element_type=F32)
            o_ref[0] = y[:, :128]
            o_ref[1] = y[:, 128:]
        elif W_next is not None:
            o_ref[...] = jnp.dot(o, wn_ref[0], preferred_element_type=F32)
        else:
            o_ref[...] = o

    return pl.pallas_call(
        body,
        grid=(N // _RB,),
        in_specs=[
            pl.BlockSpec((1, _RB, 128), lambda i: (0, i, 0)),
            pl.BlockSpec((1, _RB, 128), lambda i: (1, i, 0)),
            pl.BlockSpec((2, 128), lambda i: (0, 0)),
            pl.BlockSpec((2, 128, 512), lambda i: (0, 0, 0)),
            pl.BlockSpec((1, 512), lambda i: (0, 0)),
            pl.BlockSpec((512, 256), lambda i: (0, 0)),
            pl.BlockSpec((1, 256), lambda i: (0, 0)),
            pl.BlockSpec(wn.shape, lambda i: (0, 0, 0)),
        ],
        out_specs=out_spec,
        out_shape=out_shape,
    )(agg3, agg3, b2, M13, mb1r, M2, mb2r, wn)


def _mlp_narrow(partflat, b, M1, mb1, M2, mb2):
    p3 = partflat.reshape(2, NACC, 128)
    br = b.reshape(1, 32)
    mb1r = mb1.reshape(1, 64)
    mb2r = mb2.reshape(1, 32)

    def body(p0_ref, p1_ref, b_ref, m1_ref, mb1_ref, m2_ref, mb2_ref, o_ref):
        a = (p0_ref[0] + p1_ref[0])[:, :32]
        h = jnp.maximum(a + b_ref[0], 0.0)
        t = jnp.maximum(jnp.dot(h, m1_ref[...], preferred_element_type=F32) + mb1_ref[0], 0.0)
        o_ref[...] = jnp.dot(t, m2_ref[...], preferred_element_type=F32) + mb2_ref[0]

    return pl.pallas_call(
        body,
        grid=(N // _RB,),
        in_specs=[
            pl.BlockSpec((1, _RB, 128), lambda i: (0, i, 0)),
            pl.BlockSpec((1, _RB, 128), lambda i: (1, i, 0)),
            pl.BlockSpec((1, 32), lambda i: (0, 0)),
            pl.BlockSpec((32, 64), lambda i: (0, 0)),
            pl.BlockSpec((1, 64), lambda i: (0, 0)),
            pl.BlockSpec((64, 32), lambda i: (0, 0)),
            pl.BlockSpec((1, 32), lambda i: (0, 0)),
        ],
        out_specs=pl.BlockSpec((_RB, 32), lambda i: (i, 0)),
        out_shape=jax.ShapeDtypeStruct((N, 32), F32),
    )(p3, p3, br, M1, mb1r, M2, mb2r)


def _fc_enc(hflat, W, b):
    KB = 32000

    def body(x_ref, w_ref, b_ref, o_ref):
        @pl.when(pl.program_id(0) == 0)
        def _():
            o_ref[...] = b_ref[...]

        o_ref[...] += jnp.dot(x_ref[...], w_ref[...], preferred_element_type=F32)

    return pl.pallas_call(
        body,
        grid=(10,),
        in_specs=[
            pl.BlockSpec((1, KB), lambda i: (0, i)),
            pl.BlockSpec((KB, 64), lambda i: (i, 0)),
            pl.BlockSpec((1, 64), lambda i: (0, 0)),
        ],
        out_specs=pl.BlockSpec((1, 64), lambda i: (0, 0)),
        out_shape=jax.ShapeDtypeStruct((1, 64), F32),
    )(hflat, W, b)


def _fc_dec(z, W, b):
    NB = 32000

    def body(z_ref, w_ref, b_ref, o_ref):
        o_ref[...] = jnp.dot(z_ref[...], w_ref[...], preferred_element_type=F32) + b_ref[...]

    return pl.pallas_call(
        body,
        grid=(10,),
        in_specs=[
            pl.BlockSpec((1, 64), lambda i: (0, 0)),
            pl.BlockSpec((64, NB), lambda i: (0, i)),
            pl.BlockSpec((1, NB), lambda i: (0, i)),
        ],
        out_specs=pl.BlockSpec((1, NB), lambda i: (0, i)),
        out_shape=jax.ShapeDtypeStruct((1, 320000), F32),
    )(z, W, b)


def kernel(x, pos, enc_params, dec_params, fc_e_W, fc_e_b, fc_d_W, fc_d_b, edge_index):
    del pos
    src = edge_index[0]
    dst = edge_index[1]
    i32 = jnp.int32

    srcw = jnp.concatenate(
        [src.reshape(16, 10000), jnp.zeros((16, 112), i32)], axis=1
    ).reshape(16, 79, 128)
    srcw2 = jnp.concatenate([srcw, srcw + N], axis=0)
    dstw = jnp.concatenate(
        [dst.reshape(16, 10000), jnp.full((16, 112), N, i32)], axis=1
    ).reshape(16, 79, 128)
    dstw2 = jnp.concatenate([dstw, dstw], axis=0)
    srcn = jnp.concatenate(
        [src.reshape(32, 5000), jnp.zeros((32, 120), i32)], axis=1
    ).reshape(32, 40, 128)
    dstn = jnp.concatenate(
        [dst.reshape(32, 5000), jnp.full((32, 120), N, i32)], axis=1
    ).reshape(32, 40, 128)

    zeros_w = jnp.zeros((NACC, 128), F32)
    seg_wide = _seg_sum_make(128, 79)
    seg_narrow = _seg_sum_make(128, 40)

    def seg(yflat):
        return seg_wide(yflat, srcw2, dstw2, zeros_w)

    W1, b1, M11, mb11, M21, mb21 = enc_params[0]
    W2 = enc_params[1][0]
    W3p = jnp.concatenate(
        [enc_params[2][0], jnp.zeros((256, 96), F32)], axis=1
    )
    agg = seg(_matmul_split(x, W1))
    y2 = _mlp_wide(agg, b1, M11, mb11, M21, mb21, False, W_next=W2)
    agg = seg(y2.reshape(2 * N, 128))
    _, b2, M12, mb12, M22, mb22 = enc_params[1]
    y3 = _mlp_wide(agg, b2, M12, mb12, M22, mb22, False, W_next=W3p)
    part = seg_narrow(y3, srcn, dstn, zeros_w)
    _, b3, M13_, mb13, M23, mb23 = enc_params[2]
    h3 = _mlp_narrow(part, b3, M13_, mb13, M23, mb23)

    z = _fc_enc(h3.reshape(1, 32 * N), fc_e_W, fc_e_b.reshape(1, 64))
    d = _fc_dec(z, fc_d_W, fc_d_b.reshape(1, 320000)).reshape(N, 32)

    W4, b4, M14, mb14, M24, mb24 = dec_params[0]
    W5 = dec_params[1][0]
    W6 = dec_params[2][0]
    agg = seg(_matmul_split(d, W4))
    y5 = _mlp_wide(agg, b4, M14, mb14, M24, mb24, False, W_next=W5)
    agg = seg(y5.reshape(2 * N, 128))
    _, b5, M15, mb15, M25, mb25 = dec_params[1]
    y6 = _mlp_wide(agg, b5, M15, mb15, M25, mb25, False, W_next=W6)
    agg = seg(y6.reshape(2 * N, 128))
    _, b6, M16, mb16, M26, mb26 = dec_params[2]
    return _mlp_wide(agg, b6, M16, mb16, M26, mb26, True)


# speedup vs baseline: 2.3241x; 1.0148x over previous
import functools

import jax
import jax.numpy as jnp
from jax import lax
from jax.experimental import pallas as pl
from jax.experimental.pallas import tpu as pltpu
from jax.experimental.pallas import tpu_sc as plsc

N = 10000
E = 160000
NACC = 10240
F32 = jnp.float32


def _seg_sum_make(Cw, NCH):
    mesh = plsc.VectorSubcoreMesh(
        core_axis_name="c", subcore_axis_name="s", num_cores=2, num_subcores=16
    )

    @functools.partial(
        pl.kernel,
        out_type=jax.ShapeDtypeStruct((2 * NACC, Cw), F32),
        mesh=mesh,
        scratch_types=[
            pltpu.VMEM((NCH, 128), jnp.int32),
            pltpu.VMEM((NCH, 128), jnp.int32),
            pltpu.VMEM((128, Cw), F32),
            pltpu.VMEM_SHARED((NACC, Cw), F32),
            pltpu.SemaphoreType.DMA,
        ],
    )
    def k(y_hbm, srcp_hbm, dstp_hbm, zeros_hbm, out_hbm, src_v, dst_v, rows_v, acc, sem):
        c = lax.axis_index("c")
        s = lax.axis_index("s")
        w = c * 16 + s
        pltpu.sync_copy(srcp_hbm.at[w], src_v)
        pltpu.sync_copy(dstp_hbm.at[w], dst_v)
        pltpu.sync_copy(zeros_hbm.at[pl.ds(s * 640, 640)], acc.at[pl.ds(s * 640, 640)])
        plsc.subcore_barrier()

        def chunk(j, carry):
            pltpu.async_copy(y_hbm.at[src_v.at[j]], rows_v, sem).wait()
            pltpu.sync_copy(rows_v, acc.at[dst_v.at[j]], add=True)
            return carry

        lax.fori_loop(0, NCH, chunk, 0)
        plsc.subcore_barrier()
        pltpu.sync_copy(
            acc.at[pl.ds(s * 640, 640)], out_hbm.at[pl.ds(c * NACC + s * 640, 640)]
        )

    return k


_RB = 2000


def _matmul_split(x, W):
    K = x.shape[1]

    def body(x_ref, w_ref, o_ref):
        o_ref[...] = jnp.dot(x_ref[...], w_ref[...], preferred_element_type=F32)

    return pl.pallas_call(
        body,
        grid=(2, N // _RB),
        in_specs=[
            pl.BlockSpec((_RB, K), lambda j, i: (i, 0)),
            pl.BlockSpec((K, 128), lambda j, i: (0, j)),
        ],
        out_specs=pl.BlockSpec((_RB, 128), lambda j, i: (j * (N // _RB) + i, 0)),
        out_shape=jax.ShapeDtypeStruct((2 * N, 128), F32),
    )(x, W)


def _matmul_plain(x, W):
    K = x.shape[1]
    Co = W.shape[1]

    def body(x_ref, w_ref, o_ref):
        o_ref[...] = jnp.dot(x_ref[...], w_ref[...], preferred_element_type=F32)

    return pl.pallas_call(
        body,
        grid=(N // _RB,),
        in_specs=[
            pl.BlockSpec((_RB, K), lambda i: (i, 0)),
            pl.BlockSpec((K, Co), lambda i: (0, 0)),
        ],
        out_specs=pl.BlockSpec((_RB, Co), lambda i: (i, 0)),
        out_shape=jax.ShapeDtypeStruct((N, Co), F32),
    )(x, W)


def _mlp_wide(aggflat, b, M1, mb1, M2, mb2, final_softplus, W_next=None):
    agg3 = aggflat.reshape(2, NACC, 128)
    b2 = b.reshape(2, 128)
    M13 = M1.reshape(2, 128, 512)
    mb1r = mb1.reshape(1, 512)
    mb2r = mb2.reshape(1, 256)
    fuse_split = W_next is not None and W_next.shape[1] == 256
    if fuse_split:
        wn = W_next.reshape(1, 256, 256)
        out_spec = pl.BlockSpec((2, _RB, 128), lambda i: (0, i, 0))
        out_shape = jax.ShapeDtypeStruct((2, N, 128), F32)
    elif W_next is not None:
        wn = W_next.reshape(1, 256, W_next.shape[1])
        out_spec = pl.BlockSpec((_RB, W_next.shape[1]), lambda i: (i, 0))
        out_shape = jax.ShapeDtypeStruct((N, W_next.shape[1]), F32)
    else:
        wn = jnp.zeros((1, 1, 1), F32)
        out_spec = pl.BlockSpec((_RB, 256), lambda i: (i, 0))
        out_shape = jax.ShapeDtypeStruct((N, 256), F32)

    def body(a0_ref, a1_ref, b_ref, m1_ref, mb1_ref, m2_ref, mb2_ref, wn_ref, o_ref):
        h0 = jnp.maximum(a0_ref[0] + b_ref[0], 0.0)
        h1 = jnp.maximum(a1_ref[0] + b_ref[1], 0.0)
        t = jnp.dot(h0, m1_ref[0], preferred_element_type=F32)
        t += jnp.dot(h1, m1_ref[1], preferred_element_type=F32)
        t = jnp.maximum(t + mb1_ref[0], 0.0)
        o = jnp.dot(t, m2_ref[...], preferred_element_type=F32) + mb2_ref[0]
        if final_softplus:
            o = 10.0 * o
            o = (jnp.maximum(o, 0.0) + jnp.log1p(jnp.exp(-jnp.abs(o)))) / 10.0
        if fuse_split:
            y = jnp.dot(o, wn_ref[0], preferred_element_type=F32)
            o_ref[0] = y[:, :128]
            o_ref[1] = y[:, 128:]
        elif W_next is not None:
            o_ref[...] = jnp.dot(o, wn_ref[0], preferred_element_type=F32)
        else:
            o_ref[...] = o

    return pl.pallas_call(
        body,
        grid=(N // _RB,),
        in_specs=[
            pl.BlockSpec((1, _RB, 128), lambda i: (0, i, 0)),
            pl.BlockSpec((1, _RB, 128), lambda i: (1, i, 0)),
            pl.BlockSpec((2, 128), lambda i: (0, 0)),
            pl.BlockSpec((2, 128, 512), lambda i: (0, 0, 0)),
            pl.BlockSpec((1, 512), lambda i: (0, 0)),
            pl.BlockSpec((512, 256), lambda i: (0, 0)),
            pl.BlockSpec((1, 256), lambda i: (0, 0)),
            pl.BlockSpec(wn.shape, lambda i: (0, 0, 0)),
        ],
        out_specs=out_spec,
        out_shape=out_shape,
    )(agg3, agg3, b2, M13, mb1r, M2, mb2r, wn)


def _mlp_narrow(partflat, b, M1, mb1, M2, mb2):
    p3 = partflat.reshape(2, NACC, 128)
    br = b.reshape(1, 32)
    mb1r = mb1.reshape(1, 64)
    mb2r = mb2.reshape(1, 32)

    def body(p0_ref, p1_ref, b_ref, m1_ref, mb1_ref, m2_ref, mb2_ref, o_ref):
        a = (p0_ref[0] + p1_ref[0])[:, :32]
        h = jnp.maximum(a + b_ref[0], 0.0)
        t = jnp.maximum(jnp.dot(h, m1_ref[...], preferred_element_type=F32) + mb1_ref[0], 0.0)
        o_ref[...] = jnp.dot(t, m2_ref[...], preferred_element_type=F32) + mb2_ref[0]

    return pl.pallas_call(
        body,
        grid=(N // _RB,),
        in_specs=[
            pl.BlockSpec((1, _RB, 128), lambda i: (0, i, 0)),
            pl.BlockSpec((1, _RB, 128), lambda i: (1, i, 0)),
            pl.BlockSpec((1, 32), lambda i: (0, 0)),
            pl.BlockSpec((32, 64), lambda i: (0, 0)),
            pl.BlockSpec((1, 64), lambda i: (0, 0)),
            pl.BlockSpec((64, 32), lambda i: (0, 0)),
            pl.BlockSpec((1, 32), lambda i: (0, 0)),
        ],
        out_specs=pl.BlockSpec((_RB, 32), lambda i: (i, 0)),
        out_shape=jax.ShapeDtypeStruct((N, 32), F32),
    )(p3, p3, br, M1, mb1r, M2, mb2r)


def _fc_enc(hflat, W, b):
    KB = 32000

    def body(x_ref, w_ref, b_ref, o_ref):
        @pl.when(pl.program_id(0) == 0)
        def _():
            o_ref[...] = b_ref[...]

        o_ref[...] += jnp.dot(x_ref[...], w_ref[...], preferred_element_type=F32)

    return pl.pallas_call(
        body,
        grid=(10,),
        in_specs=[
            pl.BlockSpec((1, KB), lambda i: (0, i)),
            pl.BlockSpec((KB, 64), lambda i: (i, 0)),
            pl.BlockSpec((1, 64), lambda i: (0, 0)),
        ],
        out_specs=pl.BlockSpec((1, 64), lambda i: (0, 0)),
        out_shape=jax.ShapeDtypeStruct((1, 64), F32),
    )(hflat, W, b)


def _fc_dec(z, W, b):
    NB = 32000

    def body(z_ref, w_ref, b_ref, o_ref):
        o_ref[...] = jnp.dot(z_ref[...], w_ref[...], preferred_element_type=F32) + b_ref[...]

    return pl.pallas_call(
        body,
        grid=(10,),
        in_specs=[
            pl.BlockSpec((1, 64), lambda i: (0, 0)),
            pl.BlockSpec((64, NB), lambda i: (0, i)),
            pl.BlockSpec((1, NB), lambda i: (0, i)),
        ],
        out_specs=pl.BlockSpec((1, NB), lambda i: (0, i)),
        out_shape=jax.ShapeDtypeStruct((1, 320000), F32),
    )(z, W, b)


def kernel(x, pos, enc_params, dec_params, fc_e_W, fc_e_b, fc_d_W, fc_d_b, edge_index):
    del pos
    src = edge_index[0]
    dst = edge_index[1]
    i32 = jnp.int32

    srcw = jnp.concatenate(
        [src.reshape(16, 10000), jnp.zeros((16, 112), i32)], axis=1
    ).reshape(16, 79, 128)
    srcw2 = jnp.concatenate([srcw, srcw + N], axis=0)
    dstw = jnp.concatenate(
        [dst.reshape(16, 10000), jnp.full((16, 112), N, i32)], axis=1
    ).reshape(16, 79, 128)
    dstw2 = jnp.concatenate([dstw, dstw], axis=0)
    srcn = jnp.concatenate(
        [src.reshape(32, 5000), jnp.zeros((32, 120), i32)], axis=1
    ).reshape(32, 40, 128)
    dstn = jnp.concatenate(
        [dst.reshape(32, 5000), jnp.full((32, 120), N, i32)], axis=1
    ).reshape(32, 40, 128)

    zeros_w = jnp.zeros((NACC, 128), F32)
    seg_wide = _seg_sum_make(128, 79)
    seg_narrow = _seg_sum_make(128, 40)

    def seg(yflat):
        return seg_wide(yflat, srcw2, dstw2, zeros_w)

    W1, b1, M11, mb11, M21, mb21 = enc_params[0]
    W2 = enc_params[1][0]
    W3p = jnp.concatenate(
        [enc_params[2][0], jnp.zeros((256, 96), F32)], axis=1
    )
    agg = seg(_matmul_split(x, W1))
    y2 = _mlp_wide(agg, b1, M11, mb11, M21, mb21, False, W_next=W2)
    agg = seg(y2.reshape(2 * N, 128))
    _, b2, M12, mb12, M22, mb22 = enc_params[1]
    y3 = _mlp_wide(agg, b2, M12, mb12, M22, mb22, False, W_next=W3p)
    part = seg_narrow(y3, srcn, dstn, zeros_w)
    _, b3, M13_, mb13, M23, mb23 = enc_params[2]
    h3 = _mlp_narrow(part, b3, M13_, mb13, M23, mb23)

    z = _fc_enc(h3.reshape(1, 32 * N), fc_e_W, fc_e_b.reshape(1, 64))
    d = _fc_dec(z, fc_d_W, fc_d_b.reshape(1, 320000)).reshape(N, 32)

    W4, b4, M14, mb14, M24, mb24 = dec_params[0]
    W5 = dec_params[1][0]
    W6 = dec_params[2][0]
    agg = seg(_matmul_split(d, W4))
    y5 = _mlp_wide(agg, b4, M14, mb14, M24, mb24, False, W_next=W5)
    agg = seg(y5.reshape(2 * N, 128))
    _, b5, M15, mb15, M25, mb25 = dec_params[1]
    y6 = _mlp_wide(agg, b5, M15, mb15, M25, mb25, False, W_next=W6)
    agg = seg(y6.reshape(2 * N, 128))
    _, b6, M16, mb16, M26, mb26 = dec_params[2]
    return _mlp_wide(agg, b6, M16, mb16, M26, mb26, True)
